# Initial kernel scaffold; baseline (speedup 1.0000x reference)
import jax
import jax.numpy as jnp
from jax.experimental import pallas as pl


def kernel(x, edge_index, edge_weight, Wl1, Wr1, b1, Wl2, Wr2, b2, Wl3, Wr3, b3, Wlin, blin):
    raise NotImplementedError("write your pallas kernel here")


# speedup vs baseline: 2.5179x; 2.5179x over previous
import functools

import jax
import jax.numpy as jnp
from jax import lax
from jax.experimental import pallas as pl
from jax.experimental.pallas import tpu as pltpu
from jax.experimental.pallas import tpu_sc as plsc

NS = 16
NC = 2
K = 80
LANES = 16
HALF = 128
TN = 1024
WROWS = 16


def _sc_segsum_body(npad, rows_per_tile, *refs):
    (table_h, src_h, dst_h, w_h, s_out_h,
     sS, src_v, dst_v, w_v, rows_v, sem) = refs
    c = lax.axis_index("c")
    s = lax.axis_index("s")
    nro = npad // NS
    r0 = s * nro
    er0 = s * rows_per_tile

    def zloop(r, _):
        for g in range(HALF // LANES):
            rows_v[r, pl.ds(g * LANES, LANES)] = jnp.zeros((LANES,),
                                                           jnp.float32)
        return 0
    lax.fori_loop(0, K, zloop, 0)
    for t in range(nro // K):
        pltpu.sync_copy(rows_v, sS.at[pl.ds(r0 + t * K, K)])

    plsc.subcore_barrier()

    def window(wi, _):
        base = er0 + wi * WROWS
        pltpu.sync_copy(src_h.at[pl.ds(base, WROWS)], src_v)
        pltpu.sync_copy(dst_h.at[pl.ds(base, WROWS)], dst_v)
        pltpu.sync_copy(w_h.at[pl.ds(base, WROWS)], w_v)

        def xform(r, _):
            for g in range(K // LANES):
                v = src_v[r, pl.ds(g * LANES, LANES)]
                src_v[r, pl.ds(g * LANES, LANES)] = v + v + c
            return 0
        lax.fori_loop(0, WROWS, xform, 0)

        def chunk(ci, _):
            idx_row = src_v.at[ci]
            pltpu.async_copy(table_h.at[idx_row], rows_v, sem).wait()

            def scale(gi, _):
                wvec = w_v[ci, pl.ds(gi * LANES, LANES)]
                for l in range(LANES):
                    wi_s = wvec[l]
                    i = gi * LANES + l
                    for j in range(HALF // LANES):
                        sl = pl.ds(j * LANES, LANES)
                        rows_v[i, sl] = rows_v[i, sl] * wi_s
                return 0
            lax.fori_loop(0, K // LANES, scale, 0)

            dst_row = dst_v.at[ci]
            pltpu.sync_copy(rows_v, sS.at[dst_row], add=True)
            return 0
        lax.fori_loop(0, WROWS, chunk, 0)
        return 0
    lax.fori_loop(0, rows_per_tile // WROWS, window, 0)

    plsc.subcore_barrier()

    for t in range(nro // K):
        sl = pl.ds(r0 + t * K, K)
        pltpu.sync_copy(sS.at[sl], rows_v)
        pltpu.sync_copy(rows_v, s_out_h.at[c, sl])


@functools.lru_cache(maxsize=None)
def _make_sc_segsum(npad, rows):
    rows_per_tile = rows // NS
    mesh = plsc.VectorSubcoreMesh(core_axis_name="c", subcore_axis_name="s")
    body = functools.partial(_sc_segsum_body, npad, rows_per_tile)
    return pl.kernel(
        body,
        out_type=(jax.ShapeDtypeStruct((NC, npad, HALF), jnp.float32),),
        mesh=mesh,
        scratch_types=(
            pltpu.VMEM_SHARED((npad, HALF), jnp.float32),
            pltpu.VMEM((WROWS, K), jnp.int32),
            pltpu.VMEM((WROWS, K), jnp.int32),
            pltpu.VMEM((WROWS, K), jnp.float32),
            pltpu.VMEM((K, HALF), jnp.float32),
            pltpu.SemaphoreType.DMA,
        ),
    )


def _sc_cnt_body(npad, rows_per_tile, *refs):
    (dst_h, cnt_out_h, sCnt, dst_v, ones_v, sem) = refs
    del sem
    c = lax.axis_index("c")
    s = lax.axis_index("s")
    nro = npad // NS
    r0 = s * nro
    er0 = s * rows_per_tile

    def fill(val):
        def floop(r, _):
            for g in range(HALF // LANES):
                ones_v[r, pl.ds(g * LANES, LANES)] = jnp.full(
                    (LANES,), val, jnp.float32)
            return 0
        lax.fori_loop(0, K, floop, 0)

    @pl.when(c == 0)
    def _():
        fill(0.0)
        for t in range(nro // K):
            pltpu.sync_copy(ones_v, sCnt.at[pl.ds(r0 + t * K, K)])
        fill(1.0)

    plsc.subcore_barrier()

    @pl.when(c == 0)
    def _():
        def window(wi, _):
            base = er0 + wi * WROWS
            pltpu.sync_copy(dst_h.at[pl.ds(base, WROWS)], dst_v)

            def chunk(ci, _):
                pltpu.sync_copy(ones_v, sCnt.at[dst_v.at[ci]], add=True)
                return 0
            lax.fori_loop(0, WROWS, chunk, 0)
            return 0
        lax.fori_loop(0, rows_per_tile // WROWS, window, 0)

    plsc.subcore_barrier()

    @pl.when(c == 0)
    def _():
        for t in range(nro // K):
            sl = pl.ds(r0 + t * K, K)
            pltpu.sync_copy(sCnt.at[sl], ones_v)
            pltpu.sync_copy(ones_v, cnt_out_h.at[sl])


@functools.lru_cache(maxsize=None)
def _make_sc_cnt(npad, rows):
    mesh = plsc.VectorSubcoreMesh(core_axis_name="c", subcore_axis_name="s")
    body = functools.partial(_sc_cnt_body, npad, rows // NS)
    return pl.kernel(
        body,
        out_type=(jax.ShapeDtypeStruct((npad, HALF), jnp.float32),),
        mesh=mesh,
        scratch_types=(
            pltpu.VMEM_SHARED((npad, HALF), jnp.float32),
            pltpu.VMEM((WROWS, K), jnp.int32),
            pltpu.VMEM((K, HALF), jnp.float32),
            pltpu.SemaphoreType.DMA,
        ),
    )


def _tc_layer(h, S, cnt, Wl2, Wr, b2d):
    npad, d = h.shape

    def body(h_ref, s_ref, c_ref, wl_ref, wr_ref, b_ref, o_ref):
        r = 1.0 / jnp.maximum(c_ref[...], 1.0)
        acc = jnp.dot(s_ref[0] * r, wl_ref[0],
                      preferred_element_type=jnp.float32)
        acc = acc + jnp.dot(s_ref[1] * r, wl_ref[1],
                            preferred_element_type=jnp.float32)
        acc = acc + jnp.dot(h_ref[...], wr_ref[...],
                            preferred_element_type=jnp.float32)
        o_ref[...] = jnp.maximum(acc + b_ref[...], 0.0)

    return pl.pallas_call(
        body,
        grid=(npad // TN,),
        in_specs=[
            pl.BlockSpec((TN, d), lambda i: (i, 0)),
            pl.BlockSpec((NC, TN, HALF), lambda i: (0, i, 0)),
            pl.BlockSpec((TN, 1), lambda i: (i, 0)),
            pl.BlockSpec((NC, HALF, d), lambda i: (0, 0, 0)),
            pl.BlockSpec((d, d), lambda i: (0, 0)),
            pl.BlockSpec((1, d), lambda i: (0, 0)),
        ],
        out_specs=pl.BlockSpec((TN, d), lambda i: (i, 0)),
        out_shape=jax.ShapeDtypeStruct((npad, d), jnp.float32),
    )(h, S, cnt, Wl2, Wr, b2d)


def _tc_layer_final(h, S, cnt, Wl2, Wr, b2d, Wlin, blin2d):
    npad, d = h.shape
    dout = Wlin.shape[1]

    def body(h_ref, s_ref, c_ref, wl_ref, wr_ref, b_ref, wo_ref, bo_ref,
             o_ref):
        r = 1.0 / jnp.maximum(c_ref[...], 1.0)
        acc = jnp.dot(s_ref[0] * r, wl_ref[0],
                      preferred_element_type=jnp.float32)
        acc = acc + jnp.dot(s_ref[1] * r, wl_ref[1],
                            preferred_element_type=jnp.float32)
        acc = acc + jnp.dot(h_ref[...], wr_ref[...],
                            preferred_element_type=jnp.float32)
        h3 = jnp.maximum(acc + b_ref[...], 0.0)
        logits = jnp.dot(h3, wo_ref[...],
                         preferred_element_type=jnp.float32) + bo_ref[...]
        m = jnp.max(logits, axis=-1, keepdims=True)
        lse = m + jnp.log(jnp.sum(jnp.exp(logits - m), axis=-1,
                                  keepdims=True))
        o_ref[...] = logits - lse

    return pl.pallas_call(
        body,
        grid=(npad // TN,),
        in_specs=[
            pl.BlockSpec((TN, d), lambda i: (i, 0)),
            pl.BlockSpec((NC, TN, HALF), lambda i: (0, i, 0)),
            pl.BlockSpec((TN, 1), lambda i: (i, 0)),
            pl.BlockSpec((NC, HALF, d), lambda i: (0, 0, 0)),
            pl.BlockSpec((d, d), lambda i: (0, 0)),
            pl.BlockSpec((1, d), lambda i: (0, 0)),
            pl.BlockSpec((d, dout), lambda i: (0, 0)),
            pl.BlockSpec((1, dout), lambda i: (0, 0)),
        ],
        out_specs=pl.BlockSpec((TN, dout), lambda i: (i, 0)),
        out_shape=jax.ShapeDtypeStruct((npad, dout), jnp.float32),
    )(h, S, cnt, Wl2, Wr, b2d, Wlin, blin2d)


def kernel(x, edge_index, edge_weight, Wl1, Wr1, b1, Wl2, Wr2, b2,
           Wl3, Wr3, b3, Wlin, blin):
    n, d = x.shape
    e = edge_weight.shape[0]
    npad = -(-n // (NS * K)) * (NS * K)
    rows = -(-e // (K * NS * 8)) * (NS * 8)
    epad = rows * K - e

    src_p = jnp.concatenate([edge_index[0], jnp.zeros((epad,), jnp.int32)])
    dst_p = jnp.concatenate([edge_index[1],
                             jnp.full((epad,), n, jnp.int32)])
    w_p = jnp.concatenate([edge_weight, jnp.zeros((epad,), jnp.float32)])
    src2d = src_p.reshape(rows, K)
    dst2d = dst_p.reshape(rows, K)
    w2d = w_p.reshape(rows, K)
    xp = jnp.concatenate(
        [x, jnp.zeros((npad - n, d), jnp.float32)]) if npad > n else x

    seg = _make_sc_segsum(npad, rows)
    (cntfull,) = _make_sc_cnt(npad, rows)(dst2d)
    cnt = cntfull[:, :1]

    (S1,) = seg(xp.reshape(NC * npad, HALF), src2d, dst2d, w2d)
    h1 = _tc_layer(xp, S1, cnt, Wl1.reshape(NC, HALF, d), Wr1,
                   b1.reshape(1, d))
    (S2,) = seg(h1.reshape(NC * npad, HALF), src2d, dst2d, w2d)
    h2 = _tc_layer(h1, S2, cnt, Wl2.reshape(NC, HALF, d), Wr2,
                   b2.reshape(1, d))
    (S3,) = seg(h2.reshape(NC * npad, HALF), src2d, dst2d, w2d)
    out = _tc_layer_final(h2, S3, cnt, Wl3.reshape(NC, HALF, d), Wr3,
                          b3.reshape(1, d), Wlin,
                          blin.reshape(1, blin.shape[0]))
    return out[:n]


# speedup vs baseline: 3.0849x; 1.2252x over previous
import functools

import jax
import jax.numpy as jnp
from jax import lax
from jax.experimental import pallas as pl
from jax.experimental.pallas import tpu as pltpu
from jax.experimental.pallas import tpu_sc as plsc

NS = 16
NC = 2
K = 80
LANES = 16
HALF = 128
TN = 1024
WROWS = 16


def _sc_segsum_body(npad, rows_per_tile, *refs):
    (table_h, src_h, dst_h, w_h, s_out_h,
     sS, src_v, dst_v, w_v, rows_v, rows_v1, gsem0, gsem1, ssem0,
     ssem1) = refs
    c = lax.axis_index("c")
    s = lax.axis_index("s")
    nro = npad // NS
    r0 = s * nro
    er0 = s * rows_per_tile

    def zloop(r, _):
        for g in range(HALF // LANES):
            rows_v[r, pl.ds(g * LANES, LANES)] = jnp.zeros((LANES,),
                                                           jnp.float32)
        return 0
    lax.fori_loop(0, K, zloop, 0)
    for t in range(nro // K):
        pltpu.sync_copy(rows_v, sS.at[pl.ds(r0 + t * K, K)])

    plsc.subcore_barrier()

    def window(wi, _):
        base = er0 + wi * WROWS
        pltpu.sync_copy(src_h.at[pl.ds(base, WROWS)], src_v)
        pltpu.sync_copy(dst_h.at[pl.ds(base, WROWS)], dst_v)
        pltpu.sync_copy(w_h.at[pl.ds(base, WROWS)], w_v)

        def xform(r, _):
            for g in range(K // LANES):
                v = src_v[r, pl.ds(g * LANES, LANES)]
                src_v[r, pl.ds(g * LANES, LANES)] = v + v + c
            return 0
        lax.fori_loop(0, WROWS, xform, 0)

        def scale(buf, ci):
            def sloop(gi, _):
                wvec = w_v[ci, pl.ds(gi * LANES, LANES)]
                for l in range(LANES):
                    wi_s = wvec[l]
                    i = gi * LANES + l
                    for j in range(HALF // LANES):
                        sl = pl.ds(j * LANES, LANES)
                        buf[i, sl] = buf[i, sl] * wi_s
                return 0
            lax.fori_loop(0, K // LANES, sloop, 0)

        pltpu.async_copy(table_h.at[src_v.at[0]], rows_v, gsem0)
        pltpu.async_copy(table_h.at[src_v.at[1]], rows_v1, gsem1)

        def pair(p, _):
            c0 = 2 * p
            pltpu.make_async_copy(table_h.at[src_v.at[c0]], rows_v,
                                  gsem0).wait()
            scale(rows_v, c0)
            pltpu.async_copy(rows_v, sS.at[dst_v.at[c0]], ssem0, add=True)
            pltpu.make_async_copy(table_h.at[src_v.at[c0 + 1]], rows_v1,
                                  gsem1).wait()
            scale(rows_v1, c0 + 1)
            pltpu.async_copy(rows_v1, sS.at[dst_v.at[c0 + 1]], ssem1,
                             add=True)

            @pl.when(p < WROWS // 2 - 1)
            def _():
                pltpu.make_async_copy(rows_v, sS.at[dst_v.at[c0]],
                                      ssem0).wait()
                pltpu.async_copy(table_h.at[src_v.at[c0 + 2]], rows_v,
                                 gsem0)
                pltpu.make_async_copy(rows_v1, sS.at[dst_v.at[c0 + 1]],
                                      ssem1).wait()
                pltpu.async_copy(table_h.at[src_v.at[c0 + 3]], rows_v1,
                                 gsem1)
            return 0
        lax.fori_loop(0, WROWS // 2, pair, 0)

        pltpu.make_async_copy(rows_v, sS.at[dst_v.at[0]], ssem0).wait()
        pltpu.make_async_copy(rows_v1, sS.at[dst_v.at[1]], ssem1).wait()
        return 0
    lax.fori_loop(0, rows_per_tile // WROWS, window, 0)

    plsc.subcore_barrier()

    for t in range(nro // K):
        sl = pl.ds(r0 + t * K, K)
        pltpu.sync_copy(sS.at[sl], rows_v)
        pltpu.sync_copy(rows_v, s_out_h.at[c, sl])


@functools.lru_cache(maxsize=None)
def _make_sc_segsum(npad, rows):
    rows_per_tile = rows // NS
    mesh = plsc.VectorSubcoreMesh(core_axis_name="c", subcore_axis_name="s")
    body = functools.partial(_sc_segsum_body, npad, rows_per_tile)
    return pl.kernel(
        body,
        out_type=(jax.ShapeDtypeStruct((NC, npad, HALF), jnp.float32),),
        mesh=mesh,
        scratch_types=(
            pltpu.VMEM_SHARED((npad, HALF), jnp.float32),
            pltpu.VMEM((WROWS, K), jnp.int32),
            pltpu.VMEM((WROWS, K), jnp.int32),
            pltpu.VMEM((WROWS, K), jnp.float32),
            pltpu.VMEM((K, HALF), jnp.float32),
            pltpu.VMEM((K, HALF), jnp.float32),
            pltpu.SemaphoreType.DMA,
            pltpu.SemaphoreType.DMA,
            pltpu.SemaphoreType.DMA,
            pltpu.SemaphoreType.DMA,
        ),
    )


def _sc_cnt_body(npad, rows_per_tile, *refs):
    (dst_h, cnt_out_h, sCnt, dst_v, ones_v, sem) = refs
    del sem
    c = lax.axis_index("c")
    s = lax.axis_index("s")
    nro = npad // NS
    r0 = s * nro
    er0 = s * rows_per_tile

    def fill(val):
        def floop(r, _):
            for g in range(HALF // LANES):
                ones_v[r, pl.ds(g * LANES, LANES)] = jnp.full(
                    (LANES,), val, jnp.float32)
            return 0
        lax.fori_loop(0, K, floop, 0)

    @pl.when(c == 0)
    def _():
        fill(0.0)
        for t in range(nro // K):
            pltpu.sync_copy(ones_v, sCnt.at[pl.ds(r0 + t * K, K)])
        fill(1.0)

    plsc.subcore_barrier()

    @pl.when(c == 0)
    def _():
        def window(wi, _):
            base = er0 + wi * WROWS
            pltpu.sync_copy(dst_h.at[pl.ds(base, WROWS)], dst_v)

            def chunk(ci, _):
                pltpu.sync_copy(ones_v, sCnt.at[dst_v.at[ci]], add=True)
                return 0
            lax.fori_loop(0, WROWS, chunk, 0)
            return 0
        lax.fori_loop(0, rows_per_tile // WROWS, window, 0)

    plsc.subcore_barrier()

    @pl.when(c == 0)
    def _():
        for t in range(nro // K):
            sl = pl.ds(r0 + t * K, K)
            pltpu.sync_copy(sCnt.at[sl], ones_v)
            pltpu.sync_copy(ones_v, cnt_out_h.at[sl])


@functools.lru_cache(maxsize=None)
def _make_sc_cnt(npad, rows):
    mesh = plsc.VectorSubcoreMesh(core_axis_name="c", subcore_axis_name="s")
    body = functools.partial(_sc_cnt_body, npad, rows // NS)
    return pl.kernel(
        body,
        out_type=(jax.ShapeDtypeStruct((npad, HALF), jnp.float32),),
        mesh=mesh,
        scratch_types=(
            pltpu.VMEM_SHARED((npad, HALF), jnp.float32),
            pltpu.VMEM((WROWS, K), jnp.int32),
            pltpu.VMEM((K, HALF), jnp.float32),
            pltpu.SemaphoreType.DMA,
        ),
    )


def _tc_layer(h, S, cnt, Wl2, Wr, b2d):
    npad, d = h.shape

    def body(h_ref, s_ref, c_ref, wl_ref, wr_ref, b_ref, o_ref):
        r = 1.0 / jnp.maximum(c_ref[...], 1.0)
        acc = jnp.dot(s_ref[0] * r, wl_ref[0],
                      preferred_element_type=jnp.float32)
        acc = acc + jnp.dot(s_ref[1] * r, wl_ref[1],
                            preferred_element_type=jnp.float32)
        acc = acc + jnp.dot(h_ref[...], wr_ref[...],
                            preferred_element_type=jnp.float32)
        o_ref[...] = jnp.maximum(acc + b_ref[...], 0.0)

    return pl.pallas_call(
        body,
        grid=(npad // TN,),
        in_specs=[
            pl.BlockSpec((TN, d), lambda i: (i, 0)),
            pl.BlockSpec((NC, TN, HALF), lambda i: (0, i, 0)),
            pl.BlockSpec((TN, 1), lambda i: (i, 0)),
            pl.BlockSpec((NC, HALF, d), lambda i: (0, 0, 0)),
            pl.BlockSpec((d, d), lambda i: (0, 0)),
            pl.BlockSpec((1, d), lambda i: (0, 0)),
        ],
        out_specs=pl.BlockSpec((TN, d), lambda i: (i, 0)),
        out_shape=jax.ShapeDtypeStruct((npad, d), jnp.float32),
    )(h, S, cnt, Wl2, Wr, b2d)


def _tc_layer_final(h, S, cnt, Wl2, Wr, b2d, Wlin, blin2d):
    npad, d = h.shape
    dout = Wlin.shape[1]

    def body(h_ref, s_ref, c_ref, wl_ref, wr_ref, b_ref, wo_ref, bo_ref,
             o_ref):
        r = 1.0 / jnp.maximum(c_ref[...], 1.0)
        acc = jnp.dot(s_ref[0] * r, wl_ref[0],
                      preferred_element_type=jnp.float32)
        acc = acc + jnp.dot(s_ref[1] * r, wl_ref[1],
                            preferred_element_type=jnp.float32)
        acc = acc + jnp.dot(h_ref[...], wr_ref[...],
                            preferred_element_type=jnp.float32)
        h3 = jnp.maximum(acc + b_ref[...], 0.0)
        logits = jnp.dot(h3, wo_ref[...],
                         preferred_element_type=jnp.float32) + bo_ref[...]
        m = jnp.max(logits, axis=-1, keepdims=True)
        lse = m + jnp.log(jnp.sum(jnp.exp(logits - m), axis=-1,
                                  keepdims=True))
        o_ref[...] = logits - lse

    return pl.pallas_call(
        body,
        grid=(npad // TN,),
        in_specs=[
            pl.BlockSpec((TN, d), lambda i: (i, 0)),
            pl.BlockSpec((NC, TN, HALF), lambda i: (0, i, 0)),
            pl.BlockSpec((TN, 1), lambda i: (i, 0)),
            pl.BlockSpec((NC, HALF, d), lambda i: (0, 0, 0)),
            pl.BlockSpec((d, d), lambda i: (0, 0)),
            pl.BlockSpec((1, d), lambda i: (0, 0)),
            pl.BlockSpec((d, dout), lambda i: (0, 0)),
            pl.BlockSpec((1, dout), lambda i: (0, 0)),
        ],
        out_specs=pl.BlockSpec((TN, dout), lambda i: (i, 0)),
        out_shape=jax.ShapeDtypeStruct((npad, dout), jnp.float32),
    )(h, S, cnt, Wl2, Wr, b2d, Wlin, blin2d)


def kernel(x, edge_index, edge_weight, Wl1, Wr1, b1, Wl2, Wr2, b2,
           Wl3, Wr3, b3, Wlin, blin):
    n, d = x.shape
    e = edge_weight.shape[0]
    npad = -(-n // (NS * K)) * (NS * K)
    rows = -(-e // (K * NS * 8)) * (NS * 8)
    epad = rows * K - e

    src_p = jnp.concatenate([edge_index[0], jnp.zeros((epad,), jnp.int32)])
    dst_p = jnp.concatenate([edge_index[1],
                             jnp.full((epad,), n, jnp.int32)])
    w_p = jnp.concatenate([edge_weight, jnp.zeros((epad,), jnp.float32)])
    src2d = src_p.reshape(rows, K)
    dst2d = dst_p.reshape(rows, K)
    w2d = w_p.reshape(rows, K)
    xp = jnp.concatenate(
        [x, jnp.zeros((npad - n, d), jnp.float32)]) if npad > n else x

    seg = _make_sc_segsum(npad, rows)
    (cntfull,) = _make_sc_cnt(npad, rows)(dst2d)
    cnt = cntfull[:, :1]

    (S1,) = seg(xp.reshape(NC * npad, HALF), src2d, dst2d, w2d)
    h1 = _tc_layer(xp, S1, cnt, Wl1.reshape(NC, HALF, d), Wr1,
                   b1.reshape(1, d))
    (S2,) = seg(h1.reshape(NC * npad, HALF), src2d, dst2d, w2d)
    h2 = _tc_layer(h1, S2, cnt, Wl2.reshape(NC, HALF, d), Wr2,
                   b2.reshape(1, d))
    (S3,) = seg(h2.reshape(NC * npad, HALF), src2d, dst2d, w2d)
    out = _tc_layer_final(h2, S3, cnt, Wl3.reshape(NC, HALF, d), Wr3,
                          b3.reshape(1, d), Wlin,
                          blin.reshape(1, blin.shape[0]))
    return out[:n]
